# Initial kernel scaffold; baseline (speedup 1.0000x reference)
import jax
import jax.numpy as jnp
from jax.experimental import pallas as pl


def kernel(x, edge_index, a_i, a_j, Wx):
    raise NotImplementedError("write your pallas kernel here")


# speedup vs baseline: 16.0442x; 16.0442x over previous
import functools
import math

import jax
import jax.numpy as jnp
from jax import lax
from jax.experimental import pallas as pl
from jax.experimental.pallas import tpu as pltpu
from jax.experimental.pallas import tpu_sc as plsc

N = 10000
E = 320000
H = 128
HH = H // 2
NC = 2
NS = 16
L = 16
NW = NC * NS
EPW = E // NW
CH = 80
NCH = EPW // CH
EPT = E // NS
NCH3 = EPT // CH
DUMP = 1000
BN = 1000

_mesh = plsc.VectorSubcoreMesh(core_axis_name="c", subcore_axis_name="s")
_sc_params = pltpu.CompilerParams(needs_layout_passes=False,
                                  use_tc_tiling_on_sc=False)


def _dense_body(x_ref, wt_ref, a2_ref, h_ref, s_ref):
    xb = x_ref[...]
    hb = xb + jnp.dot(xb, wt_ref[...], preferred_element_type=jnp.float32)
    h_ref[0] = hb[:, :HH]
    h_ref[1] = hb[:, HH:]
    s_ref[...] = jnp.dot(xb, a2_ref[...], preferred_element_type=jnp.float32)


def _dense(x, wt, a2):
    return pl.pallas_call(
        _dense_body,
        grid=(N // BN,),
        in_specs=[
            pl.BlockSpec((BN, H), lambda i: (i, 0)),
            pl.BlockSpec((H, H), lambda i: (0, 0)),
            pl.BlockSpec((H, 2), lambda i: (0, 0)),
        ],
        out_specs=[
            pl.BlockSpec((NC, BN, HH), lambda i: (0, i, 0)),
            pl.BlockSpec((BN, 2), lambda i: (i, 0)),
        ],
        out_shape=[
            jax.ShapeDtypeStruct((NC, N, HH), jnp.float32),
            jax.ShapeDtypeStruct((N, 2), jnp.float32),
        ],
    )(x, wt, a2)


@functools.partial(
    pl.kernel,
    out_type=[
        jax.ShapeDtypeStruct((NW, NCH, CH), jnp.float32),
        jax.ShapeDtypeStruct((NC, 1, N), jnp.float32),
    ],
    mesh=_mesh,
    compiler_params=_sc_params,
    scratch_types=[
        pltpu.VMEM((NCH, CH), jnp.int32),
        pltpu.VMEM((NCH, CH), jnp.int32),
        pltpu.VMEM((NCH, CH), jnp.float32),
        pltpu.VMEM((N,), jnp.float32),
        pltpu.VMEM((N,), jnp.float32),
        pltpu.VMEM((1, N), jnp.float32),
        pltpu.VMEM_SHARED((1, N), jnp.float32),
    ],
)
def _edge_scores(dst_hbm, src_hbm, ei_hbm, ej_hbm,
                 eexp_hbm, seg_hbm,
                 dst_v, src_v, eexp_v, ei_v, ej_v, zero_v, seg_sh):
    c = lax.axis_index("c")
    s = lax.axis_index("s")
    wid = s * NC + c
    pltpu.sync_copy(dst_hbm.at[wid], dst_v)
    pltpu.sync_copy(src_hbm.at[wid], src_v)
    pltpu.sync_copy(ei_hbm, ei_v)
    pltpu.sync_copy(ej_hbm, ej_v)

    def _zb(i, carry):
        zero_v[0, pl.ds(i * L, L)] = jnp.zeros((L,), jnp.float32)
        return carry

    lax.fori_loop(0, N // L, _zb, 0)

    @pl.when(s == 0)
    def _():
        pltpu.sync_copy(zero_v, seg_sh)

    plsc.subcore_barrier()

    def _chunk(j, carry):
        for t in range(CH // L):
            k = t * L
            d = dst_v[j, pl.ds(k, L)]
            sr = src_v[j, pl.ds(k, L)]
            e = plsc.load_gather(ei_v, [d]) + plsc.load_gather(ej_v, [sr])
            e = jnp.where(e > 0, e, 0.01 * e)
            eexp_v[j, pl.ds(k, L)] = jnp.exp(e)
        return carry

    lax.fori_loop(0, NCH, _chunk, 0)

    def _scat(j, carry):
        pltpu.sync_copy(eexp_v.at[j], seg_sh.at[0].at[dst_v.at[j]], add=True)
        return carry

    lax.fori_loop(0, NCH, _scat, 0)

    pltpu.sync_copy(eexp_v, eexp_hbm.at[wid])
    plsc.subcore_barrier()

    @pl.when(s == 0)
    def _():
        pltpu.sync_copy(seg_sh, seg_hbm.at[c])


@functools.partial(
    pl.kernel,
    out_type=jax.ShapeDtypeStruct((NC, N, HH), jnp.float32),
    mesh=_mesh,
    compiler_params=_sc_params,
    scratch_types=[
        pltpu.VMEM((NCH3, CH), jnp.int32),
        pltpu.VMEM((NCH3, CH), jnp.int32),
        pltpu.VMEM((NCH3, CH), jnp.float32),
        pltpu.VMEM((1, N), jnp.float32),
        pltpu.VMEM((1, N), jnp.float32),
        pltpu.VMEM((CH, HH), jnp.float32),
        pltpu.VMEM_SHARED((N, HH), jnp.float32),
        pltpu.SemaphoreType.DMA,
    ],
)
def _aggregate(dst_hbm, src_hbm, eexp_hbm, seg_hbm, h_hbm, zeros_hbm,
               out_hbm,
               dst_v, src_v, eexp_v, p0_v, p1_v, rows_v, out_sh, gsem):
    c = lax.axis_index("c")
    s = lax.axis_index("s")
    pltpu.sync_copy(dst_hbm.at[s], dst_v)
    pltpu.sync_copy(src_hbm.at[s], src_v)
    pltpu.sync_copy(eexp_hbm.at[s], eexp_v)
    pltpu.sync_copy(seg_hbm.at[0], p0_v)
    pltpu.sync_copy(seg_hbm.at[1], p1_v)
    @pl.when(s < N // DUMP)
    def _():
        pltpu.sync_copy(zeros_hbm.at[pl.ds(s * DUMP, DUMP)],
                        out_sh.at[pl.ds(s * DUMP, DUMP)])

    zidx = jnp.zeros((L,), jnp.int32)

    def _al(j, carry):
        for t in range(CH // L):
            k = t * L
            d = dst_v[j, pl.ds(k, L)]
            ssum = (plsc.load_gather(p0_v, [zidx, d])
                    + plsc.load_gather(p1_v, [zidx, d]))
            eexp_v[j, pl.ds(k, L)] = eexp_v[j, pl.ds(k, L)] / (ssum + 1e-16)
        return carry

    lax.fori_loop(0, NCH3, _al, 0)
    plsc.subcore_barrier()

    hc = h_hbm.at[c]

    def _chunk(j, carry):
        pltpu.async_copy(hc.at[dst_v.at[j]], rows_v, gsem).wait()

        def _scale(eo, inner):
            for u in range(5):
                e = eo * 5 + u
                a = plsc.load_gather(
                    eexp_v,
                    [jnp.full((L,), j, jnp.int32), jnp.full((L,), e, jnp.int32)],
                )
                for r in range(HH // L):
                    rows_v[e, pl.ds(r * L, L)] = rows_v[e, pl.ds(r * L, L)] * a
            return inner

        lax.fori_loop(0, CH // 5, _scale, 0)
        pltpu.sync_copy(rows_v, out_sh.at[src_v.at[j]], add=True)
        return carry

    lax.fori_loop(0, NCH3, _chunk, 0)
    plsc.subcore_barrier()

    @pl.when(s < N // DUMP)
    def _():
        pltpu.sync_copy(out_sh.at[pl.ds(s * DUMP, DUMP)],
                        out_hbm.at[c, pl.ds(s * DUMP, DUMP)])


_INV_SQRT2 = 1.0 / math.sqrt(2.0)


def _finish_body(p_ref, o_ref):
    a = p_ref[0]
    o_ref[:, :HH] = a * 0.5 * (1.0 + lax.erf(a * _INV_SQRT2))
    b = p_ref[1]
    o_ref[:, HH:] = b * 0.5 * (1.0 + lax.erf(b * _INV_SQRT2))


def _finish(parts):
    return pl.pallas_call(
        _finish_body,
        grid=(N // BN,),
        in_specs=[pl.BlockSpec((NC, BN, HH), lambda i: (0, i, 0))],
        out_specs=pl.BlockSpec((BN, H), lambda i: (i, 0)),
        out_shape=jax.ShapeDtypeStruct((N, H), jnp.float32),
    )(parts)


@jax.jit
def _impl(x, edge_index, a_i, a_j, Wx):
    wt = Wx.T
    a2 = jnp.stack([a_i, a_j], axis=1)
    h2, scores = _dense(x, wt, a2)
    ei = scores[:, 0]
    ej = scores[:, 1]
    src = edge_index[0].reshape(NW, NCH, CH)
    dst = edge_index[1].reshape(NW, NCH, CH)
    eexp, seg = _edge_scores(dst, src, ei, ej)
    zeros = jnp.zeros((N, HH), jnp.float32)
    parts = _aggregate(dst.reshape(NS, NCH3, CH), src.reshape(NS, NCH3, CH),
                       eexp.reshape(NS, NCH3, CH), seg, h2, zeros)
    return _finish(parts)


def kernel(x, edge_index, a_i, a_j, Wx):
    return _impl(x, edge_index, a_i, a_j, Wx)


# speedup vs baseline: 20.5752x; 1.2824x over previous
import functools
import math

import jax
import jax.numpy as jnp
from jax import lax
from jax.experimental import pallas as pl
from jax.experimental.pallas import tpu as pltpu
from jax.experimental.pallas import tpu_sc as plsc

N = 10000
E = 320000
H = 128
HH = H // 2
NC = 2
NS = 16
L = 16
NW = NC * NS
EPW = E // NW
CH = 80
NCH = EPW // CH
EPT = E // NS
NCH3 = EPT // CH
DUMP = 1000
BN = 1000

_mesh = plsc.VectorSubcoreMesh(core_axis_name="c", subcore_axis_name="s")
_sc_params = pltpu.CompilerParams(needs_layout_passes=False,
                                  use_tc_tiling_on_sc=False)


def _dense_body(x_ref, wt_ref, a2_ref, h_ref, s_ref):
    xb = x_ref[...]
    hb = xb + jnp.dot(xb, wt_ref[...], preferred_element_type=jnp.float32)
    h_ref[0] = hb[:, :HH]
    h_ref[1] = hb[:, HH:]
    s_ref[...] = jnp.dot(xb, a2_ref[...], preferred_element_type=jnp.float32)


def _dense(x, wt, a2):
    return pl.pallas_call(
        _dense_body,
        grid=(N // BN,),
        in_specs=[
            pl.BlockSpec((BN, H), lambda i: (i, 0)),
            pl.BlockSpec((H, H), lambda i: (0, 0)),
            pl.BlockSpec((H, 2), lambda i: (0, 0)),
        ],
        out_specs=[
            pl.BlockSpec((NC, BN, HH), lambda i: (0, i, 0)),
            pl.BlockSpec((BN, 2), lambda i: (i, 0)),
        ],
        out_shape=[
            jax.ShapeDtypeStruct((NC, N, HH), jnp.float32),
            jax.ShapeDtypeStruct((N, 2), jnp.float32),
        ],
    )(x, wt, a2)


@functools.partial(
    pl.kernel,
    out_type=[
        jax.ShapeDtypeStruct((NW, NCH, CH), jnp.float32),
        jax.ShapeDtypeStruct((NC, 1, N), jnp.float32),
    ],
    mesh=_mesh,
    compiler_params=_sc_params,
    scratch_types=[
        pltpu.VMEM((NCH, CH), jnp.int32),
        pltpu.VMEM((NCH, CH), jnp.int32),
        pltpu.VMEM((NCH, CH), jnp.float32),
        pltpu.VMEM((N,), jnp.float32),
        pltpu.VMEM((N,), jnp.float32),
        pltpu.VMEM((1, N), jnp.float32),
        pltpu.VMEM_SHARED((1, N), jnp.float32),
    ],
)
def _edge_scores(dst_hbm, src_hbm, ei_hbm, ej_hbm,
                 eexp_hbm, seg_hbm,
                 dst_v, src_v, eexp_v, ei_v, ej_v, zero_v, seg_sh):
    c = lax.axis_index("c")
    s = lax.axis_index("s")
    wid = s * NC + c
    pltpu.sync_copy(dst_hbm.at[wid], dst_v)
    pltpu.sync_copy(src_hbm.at[wid], src_v)
    pltpu.sync_copy(ei_hbm, ei_v)
    pltpu.sync_copy(ej_hbm, ej_v)

    def _zb(i, carry):
        zero_v[0, pl.ds(i * L, L)] = jnp.zeros((L,), jnp.float32)
        return carry

    lax.fori_loop(0, N // L, _zb, 0)

    @pl.when(s == 0)
    def _():
        pltpu.sync_copy(zero_v, seg_sh)

    plsc.subcore_barrier()

    def _chunk(j, carry):
        for t in range(CH // L):
            k = t * L
            d = dst_v[j, pl.ds(k, L)]
            sr = src_v[j, pl.ds(k, L)]
            e = plsc.load_gather(ei_v, [d]) + plsc.load_gather(ej_v, [sr])
            e = jnp.where(e > 0, e, 0.01 * e)
            eexp_v[j, pl.ds(k, L)] = jnp.exp(e)
        return carry

    lax.fori_loop(0, NCH, _chunk, 0)

    def _scat(j, carry):
        pltpu.sync_copy(eexp_v.at[j], seg_sh.at[0].at[dst_v.at[j]], add=True)
        return carry

    lax.fori_loop(0, NCH, _scat, 0)

    pltpu.sync_copy(eexp_v, eexp_hbm.at[wid])
    plsc.subcore_barrier()

    @pl.when(s == 0)
    def _():
        pltpu.sync_copy(seg_sh, seg_hbm.at[c])


@functools.partial(
    pl.kernel,
    out_type=jax.ShapeDtypeStruct((NC, N, HH), jnp.float32),
    mesh=_mesh,
    compiler_params=_sc_params,
    scratch_types=[
        pltpu.VMEM((NCH3, CH), jnp.int32),
        pltpu.VMEM((NCH3, CH), jnp.int32),
        pltpu.VMEM((NCH3, CH), jnp.float32),
        pltpu.VMEM((1, N), jnp.float32),
        pltpu.VMEM((1, N), jnp.float32),
        pltpu.VMEM((2, CH, HH), jnp.float32),
        pltpu.VMEM_SHARED((N, HH), jnp.float32),
        pltpu.SemaphoreType.DMA,
        pltpu.SemaphoreType.DMA,
        pltpu.SemaphoreType.DMA,
        pltpu.SemaphoreType.DMA,
    ],
)
def _aggregate(dst_hbm, src_hbm, eexp_hbm, seg_hbm, h_hbm, zeros_hbm,
               out_hbm,
               dst_v, src_v, eexp_v, p0_v, p1_v, rows_v, out_sh,
               g0, g1, s0, s1):
    c = lax.axis_index("c")
    s = lax.axis_index("s")
    pltpu.sync_copy(dst_hbm.at[s], dst_v)
    pltpu.sync_copy(src_hbm.at[s], src_v)
    pltpu.sync_copy(eexp_hbm.at[s], eexp_v)
    pltpu.sync_copy(seg_hbm.at[0], p0_v)
    pltpu.sync_copy(seg_hbm.at[1], p1_v)
    @pl.when(s < N // DUMP)
    def _():
        pltpu.sync_copy(zeros_hbm.at[pl.ds(s * DUMP, DUMP)],
                        out_sh.at[pl.ds(s * DUMP, DUMP)])

    zidx = jnp.zeros((L,), jnp.int32)

    def _al(j, carry):
        for t in range(CH // L):
            k = t * L
            d = dst_v[j, pl.ds(k, L)]
            ssum = (plsc.load_gather(p0_v, [zidx, d])
                    + plsc.load_gather(p1_v, [zidx, d]))
            eexp_v[j, pl.ds(k, L)] = eexp_v[j, pl.ds(k, L)] / (ssum + 1e-16)
        return carry

    lax.fori_loop(0, NCH3, _al, 0)
    plsc.subcore_barrier()

    hc = h_hbm.at[c]

    def _scale_chunk(j, b):
        rows = rows_v.at[b]

        def _scale(eo, inner):
            for u in range(5):
                e = eo * 5 + u
                a = plsc.load_gather(
                    eexp_v,
                    [jnp.full((L,), j, jnp.int32), jnp.full((L,), e, jnp.int32)],
                )
                for r in range(HH // L):
                    rows[e, pl.ds(r * L, L)] = rows[e, pl.ds(r * L, L)] * a
            return inner

        lax.fori_loop(0, CH // 5, _scale, 0)

    pltpu.async_copy(hc.at[dst_v.at[0]], rows_v.at[0], g0)
    pltpu.async_copy(hc.at[dst_v.at[1]], rows_v.at[1], g1)

    def _pair(jj, carry):
        j0 = jj * 2
        j1 = j0 + 1
        pltpu.make_async_copy(hc.at[dst_v.at[j0]], rows_v.at[0], g0).wait()
        _scale_chunk(j0, 0)
        s0d = pltpu.async_copy(rows_v.at[0], out_sh.at[src_v.at[j0]], s0,
                               add=True)
        pltpu.make_async_copy(hc.at[dst_v.at[j1]], rows_v.at[1], g1).wait()
        _scale_chunk(j1, 1)
        s1d = pltpu.async_copy(rows_v.at[1], out_sh.at[src_v.at[j1]], s1,
                               add=True)
        s0d.wait()
        s1d.wait()

        @pl.when(jj + 1 < NCH3 // 2)
        def _():
            pltpu.async_copy(hc.at[dst_v.at[j0 + 2]], rows_v.at[0], g0)
            pltpu.async_copy(hc.at[dst_v.at[j1 + 2]], rows_v.at[1], g1)

        return carry

    lax.fori_loop(0, NCH3 // 2, _pair, 0)
    plsc.subcore_barrier()

    @pl.when(s < N // DUMP)
    def _():
        pltpu.sync_copy(out_sh.at[pl.ds(s * DUMP, DUMP)],
                        out_hbm.at[c, pl.ds(s * DUMP, DUMP)])


_INV_SQRT2 = 1.0 / math.sqrt(2.0)


def _finish_body(p_ref, o_ref):
    a = p_ref[0]
    o_ref[:, :HH] = a * 0.5 * (1.0 + lax.erf(a * _INV_SQRT2))
    b = p_ref[1]
    o_ref[:, HH:] = b * 0.5 * (1.0 + lax.erf(b * _INV_SQRT2))


def _finish(parts):
    return pl.pallas_call(
        _finish_body,
        grid=(N // BN,),
        in_specs=[pl.BlockSpec((NC, BN, HH), lambda i: (0, i, 0))],
        out_specs=pl.BlockSpec((BN, H), lambda i: (i, 0)),
        out_shape=jax.ShapeDtypeStruct((N, H), jnp.float32),
    )(parts)


@jax.jit
def _impl(x, edge_index, a_i, a_j, Wx):
    wt = Wx.T
    a2 = jnp.stack([a_i, a_j], axis=1)
    h2, scores = _dense(x, wt, a2)
    ei = scores[:, 0]
    ej = scores[:, 1]
    src = edge_index[0].reshape(NW, NCH, CH)
    dst = edge_index[1].reshape(NW, NCH, CH)
    eexp, seg = _edge_scores(dst, src, ei, ej)
    zeros = jnp.zeros((N, HH), jnp.float32)
    parts = _aggregate(dst.reshape(NS, NCH3, CH), src.reshape(NS, NCH3, CH),
                       eexp.reshape(NS, NCH3, CH), seg, h2, zeros)
    return _finish(parts)


def kernel(x, edge_index, a_i, a_j, Wx):
    return _impl(x, edge_index, a_i, a_j, Wx)
